# baseline (speedup 1.0000x reference)
import functools
import math

import jax
import jax.numpy as jnp
from jax import lax
from jax.experimental import pallas as pl
from jax.experimental.pallas import tpu as pltpu
from jax.experimental.pallas import tpu_sc as plsc

VOCAB = 40000
D = 300
DPAD = 384
SCALE = math.sqrt(float(D))

_info = plsc.get_sparse_core_info()
_NC, _NS = _info.num_cores, _info.num_subcores
_NW = _NC * _NS


def _scale_body(t_ref, o_ref):
    o_ref[:, :D] = t_ref[...]
    o_ref[:, D:] = jnp.zeros_like(o_ref[:, D:])


def _scale_table(table):
    rows_per_block = 1000
    grid = table.shape[0] // rows_per_block
    return pl.pallas_call(
        _scale_body,
        grid=(grid,),
        in_specs=[pl.BlockSpec((rows_per_block, D), lambda i: (i, 0))],
        out_specs=pl.BlockSpec((rows_per_block, DPAD), lambda i: (i, 0)),
        out_shape=jax.ShapeDtypeStruct((table.shape[0], DPAD), table.dtype),
    )(table)


def _make_gather(B):
    assert B % (8 * _NW) == 0
    b_per_w = B // _NW
    chunk = 128
    nchunks = b_per_w // chunk
    assert nchunks % 2 == 0
    mesh = plsc.VectorSubcoreMesh(core_axis_name="c", subcore_axis_name="s")

    @functools.partial(
        pl.kernel,
        mesh=mesh,
        out_type=jax.ShapeDtypeStruct((B, DPAD), jnp.float32),
        scratch_types=[
            pltpu.VMEM((nchunks, chunk), jnp.int32),
            pltpu.VMEM((chunk, DPAD), jnp.float32),
            pltpu.VMEM((chunk, DPAD), jnp.float32),
            pltpu.SemaphoreType.DMA,
            pltpu.SemaphoreType.DMA,
        ],
    )
    def gather(table_hbm, idx_hbm, out_hbm, idx_v, rows_a, rows_b, sem_a,
               sem_b):
        wid = lax.axis_index("s") * _NC + lax.axis_index("c")
        base = wid * b_per_w
        pltpu.sync_copy(idx_hbm.at[wid], idx_v)

        def start(c, rows_v, sem):
            pltpu.async_copy(table_hbm.at[idx_v.at[c]], rows_v, sem)

        def finish(c, rows_v, sem):
            pltpu.make_async_copy(
                table_hbm.at[idx_v.at[c]], rows_v, sem
            ).wait()
            pltpu.sync_copy(rows_v, out_hbm.at[pl.ds(base + c * chunk, chunk)])

        start(0, rows_a, sem_a)

        def body(g, _):
            c0 = 2 * g
            c1 = c0 + 1
            start(c1, rows_b, sem_b)
            finish(c0, rows_a, sem_a)

            @pl.when(c1 + 1 < nchunks)
            def _():
                start(c1 + 1, rows_a, sem_a)

            finish(c1, rows_b, sem_b)
            return _

        lax.fori_loop(0, nchunks // 2, body, None)

    return gather


def _repack_body(t_ref, o_ref):
    o_ref[...] = t_ref[:, :, :D]


def _repack(out_pad, nbatch, ntok):
    b_per_block = 16
    grid = nbatch // b_per_block
    out_pad3 = out_pad.reshape(nbatch, ntok, DPAD)
    return pl.pallas_call(
        _repack_body,
        grid=(grid,),
        in_specs=[pl.BlockSpec((b_per_block, ntok, DPAD), lambda i: (i, 0, 0))],
        out_specs=pl.BlockSpec((b_per_block, ntok, D), lambda i: (i, 0, 0)),
        out_shape=jax.ShapeDtypeStruct((nbatch, ntok, D), jnp.float32),
    )(out_pad3)


def kernel(table, x):
    B = x.shape[0] * x.shape[1]
    b_per_w = B // _NW
    chunk = 128
    idx = x.reshape(_NW, b_per_w // chunk, chunk).astype(jnp.int32)
    scaled = _scale_table(table)
    out_pad = _make_gather(B)(scaled, idx)
    out_pad3 = out_pad.reshape(x.shape[0], x.shape[1], DPAD)
    return out_pad3[:, :, :D] * jnp.float32(SCALE)


# speedup vs baseline: 1.2813x; 1.2813x over previous
import functools
import math

import jax
import jax.numpy as jnp
from jax import lax
from jax.experimental import pallas as pl
from jax.experimental.pallas import tpu as pltpu
from jax.experimental.pallas import tpu_sc as plsc

VOCAB = 40000
D = 300
DPAD = 384
SCALE = math.sqrt(float(D))

_info = plsc.get_sparse_core_info()
_NC, _NS = _info.num_cores, _info.num_subcores
_NW = _NC * _NS


def _scale_body(t_ref, o_ref):
    o_ref[:, :D] = t_ref[...] * SCALE
    o_ref[:, D:] = jnp.zeros_like(o_ref[:, D:])


def _scale_table(table):
    rows_per_block = 1000
    grid = table.shape[0] // rows_per_block
    return pl.pallas_call(
        _scale_body,
        grid=(grid,),
        in_specs=[pl.BlockSpec((rows_per_block, D), lambda i: (i, 0))],
        out_specs=pl.BlockSpec((rows_per_block, DPAD), lambda i: (i, 0)),
        out_shape=jax.ShapeDtypeStruct((table.shape[0], DPAD), table.dtype),
    )(table)


def _make_gather(B):
    assert B % (8 * _NW) == 0
    b_per_w = B // _NW
    chunk = 128
    nchunks = b_per_w // chunk
    assert nchunks % 2 == 0
    mesh = plsc.VectorSubcoreMesh(core_axis_name="c", subcore_axis_name="s")

    @functools.partial(
        pl.kernel,
        mesh=mesh,
        out_type=jax.ShapeDtypeStruct((B, DPAD), jnp.float32),
        scratch_types=[
            pltpu.VMEM((nchunks, chunk), jnp.int32),
            pltpu.VMEM((chunk, DPAD), jnp.float32),
            pltpu.VMEM((chunk, DPAD), jnp.float32),
            pltpu.SemaphoreType.DMA,
            pltpu.SemaphoreType.DMA,
        ],
    )
    def gather(table_hbm, idx_hbm, out_hbm, idx_v, rows_a, rows_b, sem_a,
               sem_b):
        wid = lax.axis_index("s") * _NC + lax.axis_index("c")
        base = wid * b_per_w
        pltpu.sync_copy(idx_hbm.at[wid], idx_v)

        def start(c, rows_v, sem):
            pltpu.async_copy(table_hbm.at[idx_v.at[c]], rows_v, sem)

        def finish(c, rows_v, sem):
            pltpu.make_async_copy(
                table_hbm.at[idx_v.at[c]], rows_v, sem
            ).wait()
            pltpu.sync_copy(rows_v, out_hbm.at[pl.ds(base + c * chunk, chunk)])

        start(0, rows_a, sem_a)

        def body(g, _):
            c0 = 2 * g
            c1 = c0 + 1
            start(c1, rows_b, sem_b)
            finish(c0, rows_a, sem_a)

            @pl.when(c1 + 1 < nchunks)
            def _():
                start(c1 + 1, rows_a, sem_a)

            finish(c1, rows_b, sem_b)
            return _

        lax.fori_loop(0, nchunks // 2, body, None)

    return gather


def kernel(table, x):
    B = x.shape[0] * x.shape[1]
    b_per_w = B // _NW
    chunk = 128
    idx = x.reshape(_NW, b_per_w // chunk, chunk).astype(jnp.int32)
    scaled = _scale_table(table)
    out_pad = _make_gather(B)(scaled, idx)
    out_pad3 = out_pad.reshape(x.shape[0], x.shape[1], DPAD)
    return out_pad3[:, :, :D]
